# baseline (speedup 1.0000x reference)
import functools

import jax
import jax.numpy as jnp
from jax import lax
from jax.experimental import pallas as pl
from jax.experimental.pallas import tpu as pltpu
from jax.experimental.pallas import tpu_sc as plsc

_L = 16


@functools.cache
def _make_center_loss_kernel(batch, num_class, vec, nw):
    rows_per_w = batch // nw
    idx_blk = 128
    n_blk = rows_per_w // idx_blk
    vregs_per_row = vec // _L

    mesh = plsc.VectorSubcoreMesh(core_axis_name="c", subcore_axis_name="s")

    @functools.partial(
        pl.kernel,
        mesh=mesh,
        out_type=jax.ShapeDtypeStruct((nw, _L), jnp.float32),
        compiler_params=pltpu.CompilerParams(use_tc_tiling_on_sc=False),
        scratch_types=[
            pltpu.VMEM((n_blk, idx_blk), jnp.int32),
            pltpu.VMEM((rows_per_w, vec), jnp.float32),
            pltpu.VMEM((rows_per_w, vec), jnp.float32),
            pltpu.VMEM((_L,), jnp.float32),
            pltpu.SemaphoreType.DMA,
        ],
    )
    def k(target_hbm, emb_hbm, centers_hbm, out_hbm,
          idx_v, rows_v, emb_v, out_v, sem):
        nc = 2
        wid = lax.axis_index("s") * nc + lax.axis_index("c")
        base = wid * n_blk

        pltpu.sync_copy(target_hbm.at[pl.ds(base, n_blk)], idx_v)
        copies = []
        for b in range(n_blk):
            copies.append(pltpu.async_copy(
                centers_hbm.at[idx_v.at[b]],
                rows_v.at[pl.ds(b * idx_blk, idx_blk)],
                sem))
        pltpu.sync_copy(emb_hbm.at[pl.ds(wid * rows_per_w, rows_per_w)], emb_v)
        for c in copies:
            c.wait()

        zero = jnp.zeros((_L,), jnp.float32)

        def body(r, accs):
            a0, a1, a2, a3 = accs
            d0 = emb_v[r, pl.ds(0 * _L, _L)] - rows_v[r, pl.ds(0 * _L, _L)]
            d1 = emb_v[r, pl.ds(1 * _L, _L)] - rows_v[r, pl.ds(1 * _L, _L)]
            d2 = emb_v[r, pl.ds(2 * _L, _L)] - rows_v[r, pl.ds(2 * _L, _L)]
            d3 = emb_v[r, pl.ds(3 * _L, _L)] - rows_v[r, pl.ds(3 * _L, _L)]
            return (a0 + d0 * d0, a1 + d1 * d1, a2 + d2 * d2, a3 + d3 * d3)

        a0, a1, a2, a3 = lax.fori_loop(
            0, rows_per_w, body, (zero, zero, zero, zero))
        out_v[...] = (a0 + a1) + (a2 + a3)
        pltpu.sync_copy(out_v, out_hbm.at[wid])

    return k


def kernel(target, vector_embedding, centers):
    batch, vec = vector_embedding.shape
    num_class = centers.shape[0]
    info = plsc.get_sparse_core_info()
    nw = info.num_cores * info.num_subcores
    k = _make_center_loss_kernel(batch, num_class, vec, nw)
    idx2d = target.reshape(batch // 128, 128)
    partials = k(idx2d, vector_embedding, centers)
    return 0.5 * jnp.sum(partials) / batch
